# baseline (speedup 1.0000x reference)
import functools

import jax
import jax.numpy as jnp
from jax import lax
from jax.experimental import pallas as pl
from jax.experimental.pallas import tpu as pltpu
from jax.experimental.pallas import tpu_sc as plsc

BATCH = 16384
K = 1000
BR = 512
NB = BATCH // BR

L = 16
NC = 2
NS = 16
NW = NC * NS
CHUNK1 = BATCH // NS
CHUNK2 = BATCH // NW
KPAD = 1024
SW = 128
R1 = CHUNK1 // SW


def _sc_body(lbl_hbm, cr_hbm, d_hbm,
             lbl1_v, ones_v, bins_v, bins_sh,
             lbl2_v, cr_v, d_v):
    cid = lax.axis_index("c")
    sid = lax.axis_index("s")
    wid = sid * NC + cid

    ones16 = jnp.ones((L,), jnp.float32)
    zeros16 = jnp.zeros((L,), jnp.float32)

    base2 = wid * CHUNK2
    pltpu.sync_copy(lbl_hbm.at[pl.ds(base2, CHUNK2)], lbl2_v)

    def _fill(j, carry):
        bins_v[pl.ds(j * L, L)] = zeros16
        return carry
    lax.fori_loop(0, KPAD // L, _fill, 0)

    def _fill1(j, carry):
        ones_v[pl.ds(j * L, L)] = ones16
        return carry
    lax.fori_loop(0, SW // L, _fill1, 0)

    base1 = sid * CHUNK1
    for j in range(R1):
        pltpu.sync_copy(lbl_hbm.at[pl.ds(base1 + j * SW, SW)], lbl1_v.at[j])

    @pl.when(sid == 0)
    def _():
        pltpu.sync_copy(bins_v, bins_sh)

    plsc.subcore_barrier()
    for j in range(R1):
        pltpu.sync_copy(ones_v, bins_sh.at[lbl1_v.at[j]], add=True)
    plsc.subcore_barrier()

    pltpu.sync_copy(bins_sh, bins_v)

    for t in range(CHUNK2 // SW):
        pltpu.sync_copy(
            bins_sh.at[lbl2_v.at[pl.ds(t * SW, SW)]],
            cr_v.at[pl.ds(t * SW, SW)],
        )
    pltpu.sync_copy(cr_v, cr_hbm.at[pl.ds(base2, CHUNK2)])

    @pl.when((cid == 0) & (sid == 0))
    def _():
        def _dd(j, a):
            return a + jnp.where(bins_v[pl.ds(j * L, L)] > 0.0, 1.0, 0.0)
        d_v[...] = lax.fori_loop(0, KPAD // L, _dd, zeros16)
        pltpu.sync_copy(d_v, d_hbm)


_sc_stats = functools.partial(
    pl.kernel,
    mesh=plsc.VectorSubcoreMesh(core_axis_name="c", subcore_axis_name="s"),
    out_type=[
        jax.ShapeDtypeStruct((BATCH,), jnp.float32),
        jax.ShapeDtypeStruct((L,), jnp.float32),
    ],
    scratch_types=[
        pltpu.VMEM((R1, SW), jnp.int32),
        pltpu.VMEM((SW,), jnp.float32),
        pltpu.VMEM((KPAD,), jnp.float32),
        pltpu.VMEM_SHARED((KPAD,), jnp.float32),
        pltpu.VMEM((CHUNK2,), jnp.int32),
        pltpu.VMEM((CHUNK2,), jnp.float32),
        pltpu.VMEM((L,), jnp.float32),
    ],
)(_sc_body)


def _tc_body(lbl_ref, ct_ref, nll_ref):
    cb = ct_ref[...]
    m = jnp.max(cb, axis=0, keepdims=True)
    s = jnp.sum(jnp.exp(cb - m), axis=0, keepdims=True)
    lse = m + jnp.log(s)

    lbl = lbl_ref[...].reshape(1, BR)
    onehot = jax.lax.broadcasted_iota(jnp.int32, (K, BR), 0) == lbl
    g = jnp.sum(jnp.where(onehot, cb, 0.0), axis=0, keepdims=True)
    nll_ref[...] = lse - g


def _fin_body(nll_ref, cr_ref, d_ref, loss_ref):
    t = jnp.sum(nll_ref[...] / cr_ref[...], keepdims=True)
    loss_ref[...] = t / jnp.sum(d_ref[...], keepdims=True)


def kernel(c, pseudo_label):
    lbl = pseudo_label.astype(jnp.int32)

    nll = pl.pallas_call(
        _tc_body,
        grid=(NB,),
        in_specs=[
            pl.BlockSpec((BR,), lambda k: (k,)),
            pl.BlockSpec((K, BR), lambda k: (0, k)),
        ],
        out_specs=pl.BlockSpec((1, BR), lambda k: (0, k)),
        out_shape=jax.ShapeDtypeStruct((1, BATCH), jnp.float32),
    )(lbl, c.T)

    cr, dv = _sc_stats(lbl)

    loss = pl.pallas_call(
        _fin_body,
        in_specs=[
            pl.BlockSpec((1, BATCH), lambda: (0, 0)),
            pl.BlockSpec((1, BATCH), lambda: (0, 0)),
            pl.BlockSpec((1, L), lambda: (0, 0)),
        ],
        out_specs=pl.BlockSpec((1, 1), lambda: (0, 0)),
        out_shape=jax.ShapeDtypeStruct((1, 1), jnp.float32),
    )(nll, cr.reshape(1, BATCH), dv.reshape(1, L))
    return loss[0, 0]


# speedup vs baseline: 1.1371x; 1.1371x over previous
import functools

import jax
import jax.numpy as jnp
from jax import lax
from jax.experimental import pallas as pl
from jax.experimental.pallas import tpu as pltpu
from jax.experimental.pallas import tpu_sc as plsc

BATCH = 16384
K = 1000
BR = 1024
NB = BATCH // BR

L = 16
NC = 2
NS = 16
NW = NC * NS
CHUNK1 = BATCH // NS
CHUNK2 = BATCH // NW
KPAD = 1024
SW = 128
R1 = CHUNK1 // SW


def _sc_body(lbl_hbm, cr_hbm, d_hbm,
             lbl1_v, ones_v, bins_v, bins_sh,
             lbl2_v, cr_v, d_v):
    cid = lax.axis_index("c")
    sid = lax.axis_index("s")
    wid = sid * NC + cid

    ones16 = jnp.ones((L,), jnp.float32)
    zeros16 = jnp.zeros((L,), jnp.float32)

    base2 = wid * CHUNK2
    pltpu.sync_copy(lbl_hbm.at[pl.ds(base2, CHUNK2)], lbl2_v)

    def _fill(j, carry):
        bins_v[pl.ds(j * L, L)] = zeros16
        return carry
    lax.fori_loop(0, KPAD // L, _fill, 0)

    def _fill1(j, carry):
        ones_v[pl.ds(j * L, L)] = ones16
        return carry
    lax.fori_loop(0, SW // L, _fill1, 0)

    base1 = sid * CHUNK1
    for j in range(R1):
        pltpu.sync_copy(lbl_hbm.at[pl.ds(base1 + j * SW, SW)], lbl1_v.at[j])

    @pl.when(sid == 0)
    def _():
        pltpu.sync_copy(bins_v, bins_sh)

    plsc.subcore_barrier()
    for j in range(R1):
        pltpu.sync_copy(ones_v, bins_sh.at[lbl1_v.at[j]], add=True)
    plsc.subcore_barrier()

    pltpu.sync_copy(bins_sh, bins_v)

    for t in range(CHUNK2 // SW):
        pltpu.sync_copy(
            bins_sh.at[lbl2_v.at[pl.ds(t * SW, SW)]],
            cr_v.at[pl.ds(t * SW, SW)],
        )
    pltpu.sync_copy(cr_v, cr_hbm.at[pl.ds(base2, CHUNK2)])

    @pl.when((cid == 0) & (sid == 0))
    def _():
        def _dd(j, a):
            return a + jnp.where(bins_v[pl.ds(j * L, L)] > 0.0, 1.0, 0.0)
        d_v[...] = lax.fori_loop(0, KPAD // L, _dd, zeros16)
        pltpu.sync_copy(d_v, d_hbm)


_sc_stats = functools.partial(
    pl.kernel,
    mesh=plsc.VectorSubcoreMesh(core_axis_name="c", subcore_axis_name="s"),
    out_type=[
        jax.ShapeDtypeStruct((BATCH,), jnp.float32),
        jax.ShapeDtypeStruct((L,), jnp.float32),
    ],
    scratch_types=[
        pltpu.VMEM((R1, SW), jnp.int32),
        pltpu.VMEM((SW,), jnp.float32),
        pltpu.VMEM((KPAD,), jnp.float32),
        pltpu.VMEM_SHARED((KPAD,), jnp.float32),
        pltpu.VMEM((CHUNK2,), jnp.int32),
        pltpu.VMEM((CHUNK2,), jnp.float32),
        pltpu.VMEM((L,), jnp.float32),
    ],
)(_sc_body)


def _tc_body(lbl_ref, ct_ref, nll_ref):
    cb = ct_ref[...]
    m = jnp.max(cb, axis=0, keepdims=True)
    s = jnp.sum(jnp.exp(cb - m), axis=0, keepdims=True)
    lse = m + jnp.log(s)

    lbl = lbl_ref[...].reshape(1, BR)
    onehot = jax.lax.broadcasted_iota(jnp.int32, (K, BR), 0) == lbl
    g = jnp.sum(jnp.where(onehot, cb, 0.0), axis=0, keepdims=True)
    nll_ref[...] = lse - g


def _fin_body(nll_ref, cr_ref, d_ref, loss_ref):
    t = jnp.sum(nll_ref[...] / cr_ref[...], keepdims=True)
    loss_ref[...] = t / jnp.sum(d_ref[...], keepdims=True)


def kernel(c, pseudo_label):
    lbl = pseudo_label.astype(jnp.int32)

    nll = pl.pallas_call(
        _tc_body,
        grid=(NB,),
        in_specs=[
            pl.BlockSpec((BR,), lambda k: (k,)),
            pl.BlockSpec((K, BR), lambda k: (0, k)),
        ],
        out_specs=pl.BlockSpec((1, BR), lambda k: (0, k)),
        out_shape=jax.ShapeDtypeStruct((1, BATCH), jnp.float32),
    )(lbl, c.T)

    cr, dv = _sc_stats(lbl)

    loss = pl.pallas_call(
        _fin_body,
        in_specs=[
            pl.BlockSpec((1, BATCH), lambda: (0, 0)),
            pl.BlockSpec((1, BATCH), lambda: (0, 0)),
            pl.BlockSpec((1, L), lambda: (0, 0)),
        ],
        out_specs=pl.BlockSpec((1, 1), lambda: (0, 0)),
        out_shape=jax.ShapeDtypeStruct((1, 1), jnp.float32),
    )(nll, cr.reshape(1, BATCH), dv.reshape(1, L))
    return loss[0, 0]


# speedup vs baseline: 1.2116x; 1.0655x over previous
import functools

import jax
import jax.numpy as jnp
from jax import lax
from jax.experimental import pallas as pl
from jax.experimental.pallas import tpu as pltpu
from jax.experimental.pallas import tpu_sc as plsc

BATCH = 16384
K = 1000
BR = 2048
NB = BATCH // BR

L = 16
NC = 2
NS = 16
NW = NC * NS
CHUNK1 = BATCH // NS
CHUNK2 = BATCH // NW
KPAD = 1024
SW = 128
R1 = CHUNK1 // SW


def _sc_body(lbl_hbm, cr_hbm, d_hbm,
             lbl1_v, ones_v, bins_v, bins_sh,
             lbl2_v, cr_v, d_v):
    cid = lax.axis_index("c")
    sid = lax.axis_index("s")
    wid = sid * NC + cid

    ones16 = jnp.ones((L,), jnp.float32)
    zeros16 = jnp.zeros((L,), jnp.float32)

    base2 = wid * CHUNK2
    pltpu.sync_copy(lbl_hbm.at[pl.ds(base2, CHUNK2)], lbl2_v)

    def _fill(j, carry):
        bins_v[pl.ds(j * L, L)] = zeros16
        return carry
    lax.fori_loop(0, KPAD // L, _fill, 0)

    def _fill1(j, carry):
        ones_v[pl.ds(j * L, L)] = ones16
        return carry
    lax.fori_loop(0, SW // L, _fill1, 0)

    base1 = sid * CHUNK1
    for j in range(R1):
        pltpu.sync_copy(lbl_hbm.at[pl.ds(base1 + j * SW, SW)], lbl1_v.at[j])

    @pl.when(sid == 0)
    def _():
        pltpu.sync_copy(bins_v, bins_sh)

    plsc.subcore_barrier()
    for j in range(R1):
        pltpu.sync_copy(ones_v, bins_sh.at[lbl1_v.at[j]], add=True)
    plsc.subcore_barrier()

    pltpu.sync_copy(bins_sh, bins_v)

    for t in range(CHUNK2 // SW):
        pltpu.sync_copy(
            bins_sh.at[lbl2_v.at[pl.ds(t * SW, SW)]],
            cr_v.at[pl.ds(t * SW, SW)],
        )
    pltpu.sync_copy(cr_v, cr_hbm.at[pl.ds(base2, CHUNK2)])

    @pl.when((cid == 0) & (sid == 0))
    def _():
        def _dd(j, a):
            return a + jnp.where(bins_v[pl.ds(j * L, L)] > 0.0, 1.0, 0.0)
        d_v[...] = lax.fori_loop(0, KPAD // L, _dd, zeros16)
        pltpu.sync_copy(d_v, d_hbm)


_sc_stats = functools.partial(
    pl.kernel,
    mesh=plsc.VectorSubcoreMesh(core_axis_name="c", subcore_axis_name="s"),
    out_type=[
        jax.ShapeDtypeStruct((BATCH,), jnp.float32),
        jax.ShapeDtypeStruct((L,), jnp.float32),
    ],
    scratch_types=[
        pltpu.VMEM((R1, SW), jnp.int32),
        pltpu.VMEM((SW,), jnp.float32),
        pltpu.VMEM((KPAD,), jnp.float32),
        pltpu.VMEM_SHARED((KPAD,), jnp.float32),
        pltpu.VMEM((CHUNK2,), jnp.int32),
        pltpu.VMEM((CHUNK2,), jnp.float32),
        pltpu.VMEM((L,), jnp.float32),
    ],
)(_sc_body)


def _tc_body(lbl_ref, ct_ref, nll_ref):
    cb = ct_ref[...]
    m = jnp.max(cb, axis=0, keepdims=True)
    s = jnp.sum(jnp.exp(cb - m), axis=0, keepdims=True)
    lse = m + jnp.log(s)

    lbl = lbl_ref[...].reshape(1, BR)
    onehot = jax.lax.broadcasted_iota(jnp.int32, (K, BR), 0) == lbl
    g = jnp.sum(jnp.where(onehot, cb, 0.0), axis=0, keepdims=True)
    nll_ref[...] = lse - g


def _fin_body(nll_ref, cr_ref, d_ref, loss_ref):
    t = jnp.sum(nll_ref[...] / cr_ref[...], keepdims=True)
    loss_ref[...] = t / jnp.sum(d_ref[...], keepdims=True)


def kernel(c, pseudo_label):
    lbl = pseudo_label.astype(jnp.int32)

    nll = pl.pallas_call(
        _tc_body,
        grid=(NB,),
        in_specs=[
            pl.BlockSpec((BR,), lambda k: (k,)),
            pl.BlockSpec((K, BR), lambda k: (0, k)),
        ],
        out_specs=pl.BlockSpec((1, BR), lambda k: (0, k)),
        out_shape=jax.ShapeDtypeStruct((1, BATCH), jnp.float32),
    )(lbl, c.T)

    cr, dv = _sc_stats(lbl)

    loss = pl.pallas_call(
        _fin_body,
        in_specs=[
            pl.BlockSpec((1, BATCH), lambda: (0, 0)),
            pl.BlockSpec((1, BATCH), lambda: (0, 0)),
            pl.BlockSpec((1, L), lambda: (0, 0)),
        ],
        out_specs=pl.BlockSpec((1, 1), lambda: (0, 0)),
        out_shape=jax.ShapeDtypeStruct((1, 1), jnp.float32),
    )(nll, cr.reshape(1, BATCH), dv.reshape(1, L))
    return loss[0, 0]
